# baseline (speedup 1.0000x reference)
import functools

import jax
import jax.numpy as jnp
from jax import lax
from jax.experimental import pallas as pl
from jax.experimental.pallas import tpu as pltpu
from jax.experimental.pallas import tpu_sc as plsc

N = 10000
E = 160000
N_PAD = 10240
NC, NS, LANES = 2, 16, 16
NW = NC * NS
C = 128
E_PAD = 163840
CH = E_PAD // (NW * C)
CHF = E_PAD // (NS * C)
EPT = E_PAD // NW
RPT = N_PAD // NS

_MESH = plsc.VectorSubcoreMesh(core_axis_name="c", subcore_axis_name="s")
_SC_PARAMS = pltpu.CompilerParams(needs_layout_passes=False,
                                  use_tc_tiling_on_sc=False)


def _wid():
    return lax.axis_index("s") * NC + lax.axis_index("c")


@functools.partial(
    pl.kernel,
    out_type=jax.ShapeDtypeStruct((NC, N_PAD), jnp.float32),
    mesh=_MESH,
    compiler_params=_SC_PARAMS,
    scratch_types=[
        pltpu.VMEM((CH, C), jnp.float32),
        pltpu.VMEM((CH, C), jnp.int32),
        pltpu.VMEM_SHARED((N_PAD,), jnp.float32),
    ],
)
def _deg_kernel(ew3, col3, z1, out, ew_v, col_v, acc_sh):
    cid = lax.axis_index("c")
    sid = lax.axis_index("s")
    w = _wid()
    pltpu.sync_copy(ew3.at[w], ew_v)
    pltpu.sync_copy(col3.at[w], col_v)
    pltpu.sync_copy(z1.at[pl.ds(0, RPT)], acc_sh.at[pl.ds(sid * RPT, RPT)])
    plsc.subcore_barrier()

    def body(j, _):
        pltpu.sync_copy(ew_v.at[j], acc_sh.at[col_v.at[j]], add=True)
        return 0

    lax.fori_loop(0, CH, body, 0)
    plsc.subcore_barrier()
    pltpu.sync_copy(acc_sh.at[pl.ds(sid * RPT, RPT)],
                    out.at[cid, pl.ds(sid * RPT, RPT)])


@functools.partial(
    pl.kernel,
    out_type=jax.ShapeDtypeStruct((E_PAD,), jnp.float32),
    mesh=_MESH,
    compiler_params=_SC_PARAMS,
    scratch_types=[
        pltpu.VMEM((N_PAD,), jnp.float32),
        pltpu.VMEM((EPT,), jnp.int32),
        pltpu.VMEM((EPT,), jnp.int32),
        pltpu.VMEM((EPT,), jnp.float32),
        pltpu.VMEM((EPT,), jnp.float32),
    ],
)
def _norm_kernel(dis, row_f, col_f, ew_f, out, dis_v, row_v, col_v, ew_v, nrm_v):
    w = _wid()
    pltpu.sync_copy(dis, dis_v)
    pltpu.sync_copy(row_f.at[pl.ds(w * EPT, EPT)], row_v)
    pltpu.sync_copy(col_f.at[pl.ds(w * EPT, EPT)], col_v)
    pltpu.sync_copy(ew_f.at[pl.ds(w * EPT, EPT)], ew_v)

    def body(i, _):
        s = pl.ds(i * LANES, LANES)
        vr = plsc.load_gather(dis_v, [row_v[s]])
        vc = plsc.load_gather(dis_v, [col_v[s]])
        nrm_v[s] = vr * vc * ew_v[s]
        return 0

    lax.fori_loop(0, EPT // LANES, body, 0)
    pltpu.sync_copy(nrm_v, out.at[pl.ds(w * EPT, EPT)])


def _scale_chunk(gbuf, nrm, j, nblk):
    def body(i, _):
        nv = nrm[j, pl.ds(i * LANES, LANES)]
        for k in range(LANES):
            e = i * LANES + k
            s = nv[k]
            for db in range(nblk):
                sl = pl.ds(db * LANES, LANES)
                gbuf[e, sl] = gbuf[e, sl] * s
        return 0

    lax.fori_loop(0, C // LANES, body, 0)


def _zero_acc(g0, acc_sh, sid, t0):
    zv = jnp.zeros((LANES,), jnp.float32)
    nblk = g0.shape[1] // LANES

    def zb(r, _):
        for db in range(nblk):
            g0[r, pl.ds(db * LANES, LANES)] = zv
        return 0

    lax.fori_loop(0, C, zb, 0)
    for k in range(RPT // C):
        pltpu.async_copy(g0, acc_sh.at[pl.ds(sid * RPT + k * C, C)], t0)


def _drain_zero_acc(g0, acc_sh, sid, t0):
    for _ in range(RPT // C):
        pltpu.make_async_copy(g0, acc_sh.at[pl.ds(sid * RPT, C)], t0).wait()


def _agg_pipe(h_sh, acc_sh, ridx, cidx, nrm, g, s, t, nblk):
    n = CH

    def wait_g(b, j):
        pltpu.make_async_copy(h_sh.at[ridx.at[j]], g[b], s[b]).wait()

    def issue_g(b, j):
        pltpu.async_copy(h_sh.at[ridx.at[j]], g[b], s[b])

    def issue_s(b, j):
        pltpu.async_copy(g[b], acc_sh.at[cidx.at[j]], t[b], add=True)

    def wait_s(b, j):
        pltpu.make_async_copy(g[b], acc_sh.at[cidx.at[j]], t[b]).wait()

    def chunk(j, b, prefetch, wait_prev):
        wait_g(b, j)
        _scale_chunk(g[b], nrm, j, nblk)
        issue_s(b, j)
        if prefetch:
            bp = (b + 2) % 4
            if wait_prev:
                wait_s(bp, j - 2)
            issue_g(bp, j + 2)

    issue_g(0, 0)
    issue_g(1, 1)
    chunk(0, 0, True, False)
    chunk(1, 1, True, False)
    chunk(2, 2, True, True)
    chunk(3, 3, True, True)

    def group(ii, _):
        j0 = 4 * ii
        chunk(j0, 0, True, True)
        chunk(j0 + 1, 1, True, True)
        chunk(j0 + 2, 2, True, True)
        chunk(j0 + 3, 3, True, True)
        return 0

    g_end = (n - 6) // 4 + 1
    lax.fori_loop(1, g_end, group, 0)
    for j in range(4 * g_end, n):
        chunk(j, j % 4, j + 2 < n, True)
    for j in range(n - 4, n):
        wait_s(j % 4, j)


def _make_agg_fs(D, DG):
    D2 = D // 2

    @functools.partial(
        pl.kernel,
        out_type=jax.ShapeDtypeStruct((N_PAD, D), jnp.float32),
        mesh=_MESH,
        compiler_params=_SC_PARAMS,
        scratch_types=[
            pltpu.VMEM((CH, C), jnp.int32),
            pltpu.VMEM((CH, C), jnp.int32),
            pltpu.VMEM((CH, C), jnp.float32),
            pltpu.VMEM((C, DG), jnp.float32),
            pltpu.VMEM((C, DG), jnp.float32),
            pltpu.VMEM((C, DG), jnp.float32),
            pltpu.VMEM((C, DG), jnp.float32),
            pltpu.VMEM_SHARED((N_PAD, DG), jnp.float32),
            pltpu.VMEM_SHARED((N_PAD, D2), jnp.float32),
            pltpu.SemaphoreType.DMA,
            pltpu.SemaphoreType.DMA,
            pltpu.SemaphoreType.DMA,
            pltpu.SemaphoreType.DMA,
            pltpu.SemaphoreType.DMA,
            pltpu.SemaphoreType.DMA,
            pltpu.SemaphoreType.DMA,
            pltpu.SemaphoreType.DMA,
        ],
    )
    def agg(h, rowt, colt, nrmt, out, ridx, cidx, nrm, g0, g1, g2, g3,
            h_sh, acc_sh, s0, s1, s2, s3, t0, t1, t2, t3):
        cid = lax.axis_index("c")
        sid = lax.axis_index("s")
        rows = pl.ds(sid * RPT, RPT)
        gbufs = [g0, g1, g2, g3]
        gsems = [s0, s1, s2, s3]
        tsems = [t0, t1, t2, t3]
        hsrc = h.at[rows, pl.ds(cid * DG, DG)]
        hs0 = pl.ds(0, CH)
        pltpu.async_copy(hsrc, h_sh.at[rows], s0)
        pltpu.async_copy(rowt.at[sid, hs0], ridx, s1)
        pltpu.async_copy(colt.at[sid, hs0], cidx, s2)
        pltpu.async_copy(nrmt.at[sid, hs0], nrm, s3)
        _zero_acc(g0, acc_sh, sid, t0)
        pltpu.make_async_copy(hsrc, h_sh.at[rows], s0).wait()
        pltpu.make_async_copy(rowt.at[sid, hs0], ridx, s1).wait()
        pltpu.make_async_copy(colt.at[sid, hs0], cidx, s2).wait()
        pltpu.make_async_copy(nrmt.at[sid, hs0], nrm, s3).wait()
        _drain_zero_acc(g0, acc_sh, sid, t0)
        plsc.subcore_barrier()
        _agg_pipe(h_sh, acc_sh, ridx, cidx, nrm, gbufs, gsems, tsems,
                  D2 // LANES)
        hs1 = pl.ds(CH, CH)
        pltpu.sync_copy(rowt.at[sid, hs1], ridx)
        pltpu.sync_copy(colt.at[sid, hs1], cidx)
        pltpu.sync_copy(nrmt.at[sid, hs1], nrm)
        _agg_pipe(h_sh, acc_sh, ridx, cidx, nrm, gbufs, gsems, tsems,
                  D2 // LANES)
        plsc.subcore_barrier()
        pltpu.sync_copy(acc_sh.at[rows],
                        out.at[rows, pl.ds(cid * D2, D2)])

    return agg


def _make_agg_es(D, DG):
    @functools.partial(
        pl.kernel,
        out_type=jax.ShapeDtypeStruct((NC, N_PAD, D), jnp.float32),
        mesh=_MESH,
        compiler_params=_SC_PARAMS,
        scratch_types=[
            pltpu.VMEM((CH, C), jnp.int32),
            pltpu.VMEM((CH, C), jnp.int32),
            pltpu.VMEM((CH, C), jnp.float32),
            pltpu.VMEM((C, DG), jnp.float32),
            pltpu.VMEM((C, DG), jnp.float32),
            pltpu.VMEM((C, DG), jnp.float32),
            pltpu.VMEM((C, DG), jnp.float32),
            pltpu.VMEM_SHARED((N_PAD, DG), jnp.float32),
            pltpu.VMEM_SHARED((N_PAD, D), jnp.float32),
            pltpu.SemaphoreType.DMA,
            pltpu.SemaphoreType.DMA,
            pltpu.SemaphoreType.DMA,
            pltpu.SemaphoreType.DMA,
            pltpu.SemaphoreType.DMA,
            pltpu.SemaphoreType.DMA,
            pltpu.SemaphoreType.DMA,
            pltpu.SemaphoreType.DMA,
        ],
    )
    def agg(h, rowt, colt, nrmt, out, ridx, cidx, nrm, g0, g1, g2, g3,
            h_sh, acc_sh, s0, s1, s2, s3, t0, t1, t2, t3):
        cid = lax.axis_index("c")
        sid = lax.axis_index("s")
        rows = pl.ds(sid * RPT, RPT)
        w = _wid()
        hsrc = h.at[rows]
        pltpu.async_copy(hsrc, h_sh.at[rows], s0)
        pltpu.async_copy(rowt.at[w], ridx, s1)
        pltpu.async_copy(colt.at[w], cidx, s2)
        pltpu.async_copy(nrmt.at[w], nrm, s3)
        _zero_acc(g0, acc_sh, sid, t0)
        pltpu.make_async_copy(hsrc, h_sh.at[rows], s0).wait()
        pltpu.make_async_copy(rowt.at[w], ridx, s1).wait()
        pltpu.make_async_copy(colt.at[w], cidx, s2).wait()
        pltpu.make_async_copy(nrmt.at[w], nrm, s3).wait()
        _drain_zero_acc(g0, acc_sh, sid, t0)
        plsc.subcore_barrier()
        _agg_pipe(h_sh, acc_sh, ridx, cidx, nrm, [g0, g1, g2, g3],
                  [s0, s1, s2, s3], [t0, t1, t2, t3], D // LANES)
        plsc.subcore_barrier()
        pltpu.sync_copy(acc_sh.at[rows], out.at[cid, rows])

    return agg


_AGG_FS = {128: _make_agg_fs(128, 64)}
_AGG_ES = {64: _make_agg_es(64, 64), 48: _make_agg_es(48, 48)}


def _dis_body(p_ref, o_ref):
    deg = p_ref[0, :] + p_ref[1, :] + 1.0
    dis = lax.rsqrt(deg)
    o_ref[0, :] = dis
    o_ref[1, :] = 1.0 / deg


def _dis_kernel(parts):
    return pl.pallas_call(
        _dis_body,
        out_shape=jax.ShapeDtypeStruct((2, N_PAD), jnp.float32),
    )(parts)


_BM = 1024


def _pad_cols(z, ow):
    k = z.shape[1]
    if ow == k:
        return z
    zpad = jnp.zeros((z.shape[0], 8), z.dtype)
    if k == 128:
        return jnp.concatenate([z[:, :64], zpad, z[:, 64:], zpad], axis=1)
    return jnp.concatenate([z, zpad], axis=1)


def _unpad_cols(h, k):
    if h.shape[1] == k:
        return h
    if k == 128:
        return jnp.concatenate([h[:, :64], h[:, 72:136]], axis=1)
    return h[:, :k]


def _mm_body(ow, x_ref, w_ref, o_ref):
    z = jnp.dot(x_ref[...], w_ref[...], preferred_element_type=jnp.float32)
    o_ref[...] = _pad_cols(z, ow)


def _matmul(x, W, ow):
    M, K = x.shape
    D = W.shape[1]
    return pl.pallas_call(
        functools.partial(_mm_body, ow),
        grid=(M // _BM,),
        in_specs=[pl.BlockSpec((_BM, K), lambda i: (i, 0)),
                  pl.BlockSpec((K, D), lambda i: (0, 0))],
        out_specs=pl.BlockSpec((_BM, ow), lambda i: (i, 0)),
        out_shape=jax.ShapeDtypeStruct((M, ow), jnp.float32),
    )(x, W)


def _comb1_mm_body(ow, p_ref, h_ref, d2_ref, b_ref, w_ref, o_ref):
    k = p_ref.shape[1]
    a = p_ref[...] + d2_ref[...] * _unpad_cols(h_ref[...], k) + b_ref[...]
    a = jnp.maximum(a, 0.0)
    z = jnp.dot(a, w_ref[...], preferred_element_type=jnp.float32)
    o_ref[...] = _pad_cols(z, ow)


def _combine1_matmul(p, h, dis2, b, W, ow):
    M, K = p.shape
    KH = h.shape[1]
    D = W.shape[1]
    return pl.pallas_call(
        functools.partial(_comb1_mm_body, ow),
        grid=(M // _BM,),
        in_specs=[pl.BlockSpec((_BM, K), lambda i: (i, 0)),
                  pl.BlockSpec((_BM, KH), lambda i: (i, 0)),
                  pl.BlockSpec((_BM, 1), lambda i: (i, 0)),
                  pl.BlockSpec((1, K), lambda i: (0, 0)),
                  pl.BlockSpec((K, D), lambda i: (0, 0))],
        out_specs=pl.BlockSpec((_BM, ow), lambda i: (i, 0)),
        out_shape=jax.ShapeDtypeStruct((M, ow), jnp.float32),
    )(p, h, dis2, b.reshape(1, K), W)


def _comb2_mm_body(ow, p0_ref, p1_ref, h_ref, d2_ref, b_ref, w_ref, o_ref):
    k = p0_ref.shape[1]
    a = (p0_ref[...] + p1_ref[...]
         + d2_ref[...] * _unpad_cols(h_ref[...], k) + b_ref[...])
    a = jnp.maximum(a, 0.0)
    z = jnp.dot(a, w_ref[...], preferred_element_type=jnp.float32)
    o_ref[...] = _pad_cols(z, ow)


def _combine2_matmul(p0, p1, h, dis2, b, W, ow):
    M, K = p0.shape
    KH = h.shape[1]
    D = W.shape[1]
    blk = pl.BlockSpec((_BM, K), lambda i: (i, 0))
    return pl.pallas_call(
        functools.partial(_comb2_mm_body, ow),
        grid=(M // _BM,),
        in_specs=[blk, blk,
                  pl.BlockSpec((_BM, KH), lambda i: (i, 0)),
                  pl.BlockSpec((_BM, 1), lambda i: (i, 0)),
                  pl.BlockSpec((1, K), lambda i: (0, 0)),
                  pl.BlockSpec((K, D), lambda i: (0, 0))],
        out_specs=pl.BlockSpec((_BM, ow), lambda i: (i, 0)),
        out_shape=jax.ShapeDtypeStruct((M, ow), jnp.float32),
    )(p0, p1, h, dis2, b.reshape(1, K), W)


def _final_body(p0_ref, p1_ref, h_ref, d2_ref, b_ref, o_ref):
    z = p0_ref[...] + p1_ref[...] + d2_ref[...] * h_ref[...] + b_ref[...]
    mask = lax.broadcasted_iota(jnp.int32, z.shape, 1) < 40
    zm = jnp.where(mask, z, -jnp.inf)
    mx = jnp.max(zm, axis=1, keepdims=True)
    ex = jnp.where(mask, jnp.exp(z - mx), 0.0)
    lse = jnp.log(jnp.sum(ex, axis=1, keepdims=True))
    o_ref[...] = (z - mx - lse)[:, :40]


def _final(p0, p1, h, dis2, b):
    M, K = h.shape
    blk = pl.BlockSpec((_BM, K), lambda i: (i, 0))
    return pl.pallas_call(
        _final_body,
        grid=(M // _BM,),
        in_specs=[blk, blk, blk,
                  pl.BlockSpec((_BM, 1), lambda i: (i, 0)),
                  pl.BlockSpec((1, K), lambda i: (0, 0))],
        out_specs=pl.BlockSpec((_BM, 40), lambda i: (i, 0)),
        out_shape=jax.ShapeDtypeStruct((M, 40), jnp.float32),
    )(p0, p1, h, dis2, b.reshape(1, K))


def kernel(x, edge_index, edge_attr, W1, b1, W2, b2, W3, b3, W4, b4):
    pad = E_PAD - E
    row = jnp.concatenate([edge_index[0].astype(jnp.int32),
                           jnp.zeros((pad,), jnp.int32)])
    col = jnp.concatenate([edge_index[1].astype(jnp.int32),
                           jnp.zeros((pad,), jnp.int32)])
    ew = jnp.concatenate([edge_attr.astype(jnp.float32),
                          jnp.zeros((pad,), jnp.float32)])
    row3 = row.reshape(NW, CH, C)
    col3 = col.reshape(NW, CH, C)
    ew3 = ew.reshape(NW, CH, C)
    rowf = row.reshape(NS, CHF, C)
    colf = col.reshape(NS, CHF, C)
    z1 = jnp.zeros((N_PAD,), jnp.float32)
    xp = jnp.pad(x, ((0, N_PAD - N), (0, 0)))

    parts = _deg_kernel(ew3, col3, z1)
    dd = _dis_kernel(parts)
    dis2 = dd[1].reshape(N_PAD, 1)

    nrm = _norm_kernel(dd[0], row, col, ew)
    nrm3 = nrm.reshape(NW, CH, C)
    nrmf = nrm.reshape(NS, CHF, C)

    h1 = _matmul(xp, W1, 128)
    p = _AGG_FS[128](h1, rowf, colf, nrmf)
    h2 = _combine1_matmul(p, h1, dis2, b1, W2, 128)
    p = _AGG_FS[128](h2, rowf, colf, nrmf)
    h3 = _combine1_matmul(p, h2, dis2, b2, W3, 64)
    p = _AGG_ES[64](h3, row3, col3, nrm3)
    W4p = jnp.pad(W4, ((0, 0), (0, 8)))
    h4 = _combine2_matmul(p[0], p[1], h3, dis2, b3, W4p, 48)
    p = _AGG_ES[48](h4, row3, col3, nrm3)
    out = _final(p[0], p[1], h4, dis2, jnp.pad(b4, (0, 8)))
    return out[:N]


# speedup vs baseline: 1.0606x; 1.0606x over previous
import functools

import jax
import jax.numpy as jnp
from jax import lax
from jax.experimental import pallas as pl
from jax.experimental.pallas import tpu as pltpu
from jax.experimental.pallas import tpu_sc as plsc

N = 10000
E = 160000
N_PAD = 10240
NC, NS, LANES = 2, 16, 16
NW = NC * NS
C = 128
E_PAD = 163840
CH = E_PAD // (NW * C)
CHF = E_PAD // (NS * C)
EPT = E_PAD // NW
RPT = N_PAD // NS

_MESH = plsc.VectorSubcoreMesh(core_axis_name="c", subcore_axis_name="s")
_SC_PARAMS = pltpu.CompilerParams(needs_layout_passes=False,
                                  use_tc_tiling_on_sc=False)


def _wid():
    return lax.axis_index("s") * NC + lax.axis_index("c")


@functools.partial(
    pl.kernel,
    out_type=jax.ShapeDtypeStruct((NC, N_PAD), jnp.float32),
    mesh=_MESH,
    compiler_params=_SC_PARAMS,
    scratch_types=[
        pltpu.VMEM((CH, C), jnp.float32),
        pltpu.VMEM((CH, C), jnp.int32),
        pltpu.VMEM_SHARED((N_PAD,), jnp.float32),
    ],
)
def _deg_kernel(ew3, col3, z1, out, ew_v, col_v, acc_sh):
    cid = lax.axis_index("c")
    sid = lax.axis_index("s")
    w = _wid()
    pltpu.sync_copy(ew3.at[w], ew_v)
    pltpu.sync_copy(col3.at[w], col_v)
    pltpu.sync_copy(z1.at[pl.ds(0, RPT)], acc_sh.at[pl.ds(sid * RPT, RPT)])
    plsc.subcore_barrier()

    def body(j, _):
        pltpu.sync_copy(ew_v.at[j], acc_sh.at[col_v.at[j]], add=True)
        return 0

    lax.fori_loop(0, CH, body, 0)
    plsc.subcore_barrier()
    pltpu.sync_copy(acc_sh.at[pl.ds(sid * RPT, RPT)],
                    out.at[cid, pl.ds(sid * RPT, RPT)])


@functools.partial(
    pl.kernel,
    out_type=jax.ShapeDtypeStruct((E_PAD,), jnp.float32),
    mesh=_MESH,
    compiler_params=_SC_PARAMS,
    scratch_types=[
        pltpu.VMEM((N_PAD,), jnp.float32),
        pltpu.VMEM((EPT,), jnp.int32),
        pltpu.VMEM((EPT,), jnp.int32),
        pltpu.VMEM((EPT,), jnp.float32),
        pltpu.VMEM((EPT,), jnp.float32),
    ],
)
def _norm_kernel(dis, row_f, col_f, ew_f, out, dis_v, row_v, col_v, ew_v, nrm_v):
    w = _wid()
    pltpu.sync_copy(dis, dis_v)
    pltpu.sync_copy(row_f.at[pl.ds(w * EPT, EPT)], row_v)
    pltpu.sync_copy(col_f.at[pl.ds(w * EPT, EPT)], col_v)
    pltpu.sync_copy(ew_f.at[pl.ds(w * EPT, EPT)], ew_v)

    def body(i, _):
        s = pl.ds(i * LANES, LANES)
        vr = plsc.load_gather(dis_v, [row_v[s]])
        vc = plsc.load_gather(dis_v, [col_v[s]])
        nrm_v[s] = vr * vc * ew_v[s]
        return 0

    lax.fori_loop(0, EPT // LANES, body, 0)
    pltpu.sync_copy(nrm_v, out.at[pl.ds(w * EPT, EPT)])


def _scale_chunk(gbuf, nrm, j, nblk):
    def body(i, _):
        nv = nrm[j, pl.ds(i * LANES, LANES)]
        for k in range(LANES):
            e = i * LANES + k
            s = nv[k]
            for db in range(nblk):
                sl = pl.ds(db * LANES, LANES)
                gbuf[e, sl] = gbuf[e, sl] * s
        return 0

    lax.fori_loop(0, C // LANES, body, 0)


def _agg_pipe(h_sh, acc_sh, ridx, cidx, nrm, g, s, t, nblk):
    n = CH

    def wait_g(b, j):
        pltpu.make_async_copy(h_sh.at[ridx.at[j]], g[b], s[b]).wait()

    def issue_g(b, j):
        pltpu.async_copy(h_sh.at[ridx.at[j]], g[b], s[b])

    def issue_s(b, j):
        pltpu.async_copy(g[b], acc_sh.at[cidx.at[j]], t[b], add=True)

    def wait_s(b, j):
        pltpu.make_async_copy(g[b], acc_sh.at[cidx.at[j]], t[b]).wait()

    def chunk(j, b, prefetch, wait_prev):
        wait_g(b, j)
        _scale_chunk(g[b], nrm, j, nblk)
        issue_s(b, j)
        if prefetch:
            bp = (b + 2) % 4
            if wait_prev:
                wait_s(bp, j - 2)
            issue_g(bp, j + 2)

    issue_g(0, 0)
    issue_g(1, 1)
    chunk(0, 0, True, False)
    chunk(1, 1, True, False)
    chunk(2, 2, True, True)
    chunk(3, 3, True, True)

    def group(ii, _):
        j0 = 4 * ii
        chunk(j0, 0, True, True)
        chunk(j0 + 1, 1, True, True)
        chunk(j0 + 2, 2, True, True)
        chunk(j0 + 3, 3, True, True)
        return 0

    g_end = (n - 6) // 4 + 1
    lax.fori_loop(1, g_end, group, 0)
    for j in range(4 * g_end, n):
        chunk(j, j % 4, j + 2 < n, True)
    for j in range(n - 4, n):
        wait_s(j % 4, j)


def _make_agg_fs(D, DG):
    D2 = D // 2

    @functools.partial(
        pl.kernel,
        out_type=jax.ShapeDtypeStruct((N_PAD, D), jnp.float32),
        mesh=_MESH,
        compiler_params=_SC_PARAMS,
        scratch_types=[
            pltpu.VMEM((CH, C), jnp.int32),
            pltpu.VMEM((CH, C), jnp.int32),
            pltpu.VMEM((CH, C), jnp.float32),
            pltpu.VMEM((C, DG), jnp.float32),
            pltpu.VMEM((C, DG), jnp.float32),
            pltpu.VMEM((C, DG), jnp.float32),
            pltpu.VMEM((C, DG), jnp.float32),
            pltpu.VMEM_SHARED((N_PAD, DG), jnp.float32),
            pltpu.VMEM_SHARED((N_PAD, D2), jnp.float32),
            pltpu.SemaphoreType.DMA,
            pltpu.SemaphoreType.DMA,
            pltpu.SemaphoreType.DMA,
            pltpu.SemaphoreType.DMA,
            pltpu.SemaphoreType.DMA,
            pltpu.SemaphoreType.DMA,
            pltpu.SemaphoreType.DMA,
            pltpu.SemaphoreType.DMA,
        ],
    )
    def agg(h, rowt, colt, nrmt, z2, out, ridx, cidx, nrm, g0, g1, g2, g3,
            h_sh, acc_sh, s0, s1, s2, s3, t0, t1, t2, t3):
        cid = lax.axis_index("c")
        sid = lax.axis_index("s")
        rows = pl.ds(sid * RPT, RPT)
        gbufs = [g0, g1, g2, g3]
        gsems = [s0, s1, s2, s3]
        tsems = [t0, t1, t2, t3]
        hsrc = h.at[rows, pl.ds(cid * DG, DG)]
        hs0 = pl.ds(0, CH)
        pltpu.async_copy(hsrc, h_sh.at[rows], s0)
        pltpu.async_copy(rowt.at[sid, hs0], ridx, s1)
        pltpu.async_copy(colt.at[sid, hs0], cidx, s2)
        pltpu.async_copy(nrmt.at[sid, hs0], nrm, s3)
        pltpu.async_copy(z2, acc_sh.at[rows], t0)
        pltpu.make_async_copy(hsrc, h_sh.at[rows], s0).wait()
        pltpu.make_async_copy(rowt.at[sid, hs0], ridx, s1).wait()
        pltpu.make_async_copy(colt.at[sid, hs0], cidx, s2).wait()
        pltpu.make_async_copy(nrmt.at[sid, hs0], nrm, s3).wait()
        pltpu.make_async_copy(z2, acc_sh.at[rows], t0).wait()
        plsc.subcore_barrier()
        _agg_pipe(h_sh, acc_sh, ridx, cidx, nrm, gbufs, gsems, tsems,
                  D2 // LANES)
        hs1 = pl.ds(CH, CH)
        pltpu.sync_copy(rowt.at[sid, hs1], ridx)
        pltpu.sync_copy(colt.at[sid, hs1], cidx)
        pltpu.sync_copy(nrmt.at[sid, hs1], nrm)
        _agg_pipe(h_sh, acc_sh, ridx, cidx, nrm, gbufs, gsems, tsems,
                  D2 // LANES)
        plsc.subcore_barrier()
        pltpu.sync_copy(acc_sh.at[rows],
                        out.at[rows, pl.ds(cid * D2, D2)])

    return agg


def _make_agg_es(D, DG):
    @functools.partial(
        pl.kernel,
        out_type=jax.ShapeDtypeStruct((NC, N_PAD, D), jnp.float32),
        mesh=_MESH,
        compiler_params=_SC_PARAMS,
        scratch_types=[
            pltpu.VMEM((CH, C), jnp.int32),
            pltpu.VMEM((CH, C), jnp.int32),
            pltpu.VMEM((CH, C), jnp.float32),
            pltpu.VMEM((C, DG), jnp.float32),
            pltpu.VMEM((C, DG), jnp.float32),
            pltpu.VMEM((C, DG), jnp.float32),
            pltpu.VMEM((C, DG), jnp.float32),
            pltpu.VMEM_SHARED((N_PAD, DG), jnp.float32),
            pltpu.VMEM_SHARED((N_PAD, D), jnp.float32),
            pltpu.SemaphoreType.DMA,
            pltpu.SemaphoreType.DMA,
            pltpu.SemaphoreType.DMA,
            pltpu.SemaphoreType.DMA,
            pltpu.SemaphoreType.DMA,
            pltpu.SemaphoreType.DMA,
            pltpu.SemaphoreType.DMA,
            pltpu.SemaphoreType.DMA,
        ],
    )
    def agg(h, rowt, colt, nrmt, z2, out, ridx, cidx, nrm, g0, g1, g2, g3,
            h_sh, acc_sh, s0, s1, s2, s3, t0, t1, t2, t3):
        cid = lax.axis_index("c")
        sid = lax.axis_index("s")
        rows = pl.ds(sid * RPT, RPT)
        w = _wid()
        hsrc = h.at[rows]
        pltpu.async_copy(hsrc, h_sh.at[rows], s0)
        pltpu.async_copy(rowt.at[w], ridx, s1)
        pltpu.async_copy(colt.at[w], cidx, s2)
        pltpu.async_copy(nrmt.at[w], nrm, s3)
        pltpu.async_copy(z2, acc_sh.at[rows], t0)
        pltpu.make_async_copy(hsrc, h_sh.at[rows], s0).wait()
        pltpu.make_async_copy(rowt.at[w], ridx, s1).wait()
        pltpu.make_async_copy(colt.at[w], cidx, s2).wait()
        pltpu.make_async_copy(nrmt.at[w], nrm, s3).wait()
        pltpu.make_async_copy(z2, acc_sh.at[rows], t0).wait()
        plsc.subcore_barrier()
        _agg_pipe(h_sh, acc_sh, ridx, cidx, nrm, [g0, g1, g2, g3],
                  [s0, s1, s2, s3], [t0, t1, t2, t3], D // LANES)
        plsc.subcore_barrier()
        pltpu.sync_copy(acc_sh.at[rows], out.at[cid, rows])

    return agg


_AGG_FS = {128: _make_agg_fs(128, 64)}
_AGG_ES = {64: _make_agg_es(64, 64), 48: _make_agg_es(48, 48)}


def _dis_body(p_ref, o_ref):
    deg = p_ref[0, :] + p_ref[1, :] + 1.0
    dis = lax.rsqrt(deg)
    o_ref[0, :] = dis
    o_ref[1, :] = 1.0 / deg


def _dis_kernel(parts):
    return pl.pallas_call(
        _dis_body,
        out_shape=jax.ShapeDtypeStruct((2, N_PAD), jnp.float32),
    )(parts)


_BM = 1024


def _pad_cols(z, ow):
    k = z.shape[1]
    if ow == k:
        return z
    zpad = jnp.zeros((z.shape[0], 8), z.dtype)
    if k == 128:
        return jnp.concatenate([z[:, :64], zpad, z[:, 64:], zpad], axis=1)
    return jnp.concatenate([z, zpad], axis=1)


def _unpad_cols(h, k):
    if h.shape[1] == k:
        return h
    if k == 128:
        return jnp.concatenate([h[:, :64], h[:, 72:136]], axis=1)
    return h[:, :k]


def _mm_body(ow, x_ref, w_ref, o_ref):
    z = jnp.dot(x_ref[...], w_ref[...], preferred_element_type=jnp.float32)
    o_ref[...] = _pad_cols(z, ow)


def _matmul(x, W, ow):
    M, K = x.shape
    D = W.shape[1]
    return pl.pallas_call(
        functools.partial(_mm_body, ow),
        grid=(M // _BM,),
        in_specs=[pl.BlockSpec((_BM, K), lambda i: (i, 0)),
                  pl.BlockSpec((K, D), lambda i: (0, 0))],
        out_specs=pl.BlockSpec((_BM, ow), lambda i: (i, 0)),
        out_shape=jax.ShapeDtypeStruct((M, ow), jnp.float32),
    )(x, W)


def _comb1_mm_body(ow, p_ref, h_ref, d2_ref, b_ref, w_ref, o_ref):
    k = p_ref.shape[1]
    a = p_ref[...] + d2_ref[...] * _unpad_cols(h_ref[...], k) + b_ref[...]
    a = jnp.maximum(a, 0.0)
    z = jnp.dot(a, w_ref[...], preferred_element_type=jnp.float32)
    o_ref[...] = _pad_cols(z, ow)


def _combine1_matmul(p, h, dis2, b, W, ow):
    M, K = p.shape
    KH = h.shape[1]
    D = W.shape[1]
    return pl.pallas_call(
        functools.partial(_comb1_mm_body, ow),
        grid=(M // _BM,),
        in_specs=[pl.BlockSpec((_BM, K), lambda i: (i, 0)),
                  pl.BlockSpec((_BM, KH), lambda i: (i, 0)),
                  pl.BlockSpec((_BM, 1), lambda i: (i, 0)),
                  pl.BlockSpec((1, K), lambda i: (0, 0)),
                  pl.BlockSpec((K, D), lambda i: (0, 0))],
        out_specs=pl.BlockSpec((_BM, ow), lambda i: (i, 0)),
        out_shape=jax.ShapeDtypeStruct((M, ow), jnp.float32),
    )(p, h, dis2, b.reshape(1, K), W)


def _comb2_mm_body(ow, p0_ref, p1_ref, h_ref, d2_ref, b_ref, w_ref, o_ref):
    k = p0_ref.shape[1]
    a = (p0_ref[...] + p1_ref[...]
         + d2_ref[...] * _unpad_cols(h_ref[...], k) + b_ref[...])
    a = jnp.maximum(a, 0.0)
    z = jnp.dot(a, w_ref[...], preferred_element_type=jnp.float32)
    o_ref[...] = _pad_cols(z, ow)


def _combine2_matmul(p0, p1, h, dis2, b, W, ow):
    M, K = p0.shape
    KH = h.shape[1]
    D = W.shape[1]
    blk = pl.BlockSpec((_BM, K), lambda i: (i, 0))
    return pl.pallas_call(
        functools.partial(_comb2_mm_body, ow),
        grid=(M // _BM,),
        in_specs=[blk, blk,
                  pl.BlockSpec((_BM, KH), lambda i: (i, 0)),
                  pl.BlockSpec((_BM, 1), lambda i: (i, 0)),
                  pl.BlockSpec((1, K), lambda i: (0, 0)),
                  pl.BlockSpec((K, D), lambda i: (0, 0))],
        out_specs=pl.BlockSpec((_BM, ow), lambda i: (i, 0)),
        out_shape=jax.ShapeDtypeStruct((M, ow), jnp.float32),
    )(p0, p1, h, dis2, b.reshape(1, K), W)


def _final_body(p0_ref, p1_ref, h_ref, d2_ref, b_ref, o_ref):
    z = p0_ref[...] + p1_ref[...] + d2_ref[...] * h_ref[...] + b_ref[...]
    mask = lax.broadcasted_iota(jnp.int32, z.shape, 1) < 40
    zm = jnp.where(mask, z, -jnp.inf)
    mx = jnp.max(zm, axis=1, keepdims=True)
    ex = jnp.where(mask, jnp.exp(z - mx), 0.0)
    lse = jnp.log(jnp.sum(ex, axis=1, keepdims=True))
    o_ref[...] = (z - mx - lse)[:, :40]


def _final(p0, p1, h, dis2, b):
    M, K = h.shape
    blk = pl.BlockSpec((_BM, K), lambda i: (i, 0))
    return pl.pallas_call(
        _final_body,
        grid=(M // _BM,),
        in_specs=[blk, blk, blk,
                  pl.BlockSpec((_BM, 1), lambda i: (i, 0)),
                  pl.BlockSpec((1, K), lambda i: (0, 0))],
        out_specs=pl.BlockSpec((_BM, 40), lambda i: (i, 0)),
        out_shape=jax.ShapeDtypeStruct((M, 40), jnp.float32),
    )(p0, p1, h, dis2, b.reshape(1, K))


def kernel(x, edge_index, edge_attr, W1, b1, W2, b2, W3, b3, W4, b4):
    pad = E_PAD - E
    row = jnp.concatenate([edge_index[0].astype(jnp.int32),
                           jnp.zeros((pad,), jnp.int32)])
    col = jnp.concatenate([edge_index[1].astype(jnp.int32),
                           jnp.zeros((pad,), jnp.int32)])
    ew = jnp.concatenate([edge_attr.astype(jnp.float32),
                          jnp.zeros((pad,), jnp.float32)])
    row3 = row.reshape(NW, CH, C)
    col3 = col.reshape(NW, CH, C)
    ew3 = ew.reshape(NW, CH, C)
    rowf = row.reshape(NS, CHF, C)
    colf = col.reshape(NS, CHF, C)
    z1 = jnp.zeros((N_PAD,), jnp.float32)
    xp = jnp.pad(x, ((0, N_PAD - N), (0, 0)))

    parts = _deg_kernel(ew3, col3, z1)
    dd = _dis_kernel(parts)
    dis2 = dd[1].reshape(N_PAD, 1)

    nrm = _norm_kernel(dd[0], row, col, ew)
    nrm3 = nrm.reshape(NW, CH, C)
    nrmf = nrm.reshape(NS, CHF, C)

    z64 = jnp.zeros((RPT, 64), jnp.float32)
    z48 = jnp.zeros((RPT, 48), jnp.float32)

    h1 = _matmul(xp, W1, 128)
    p = _AGG_FS[128](h1, rowf, colf, nrmf, z64)
    h2 = _combine1_matmul(p, h1, dis2, b1, W2, 128)
    p = _AGG_FS[128](h2, rowf, colf, nrmf, z64)
    h3 = _combine1_matmul(p, h2, dis2, b2, W3, 64)
    p = _AGG_ES[64](h3, row3, col3, nrm3, z64)
    W4p = jnp.pad(W4, ((0, 0), (0, 8)))
    h4 = _combine2_matmul(p[0], p[1], h3, dis2, b3, W4p, 48)
    p = _AGG_ES[48](h4, row3, col3, nrm3, z48)
    out = _final(p[0], p[1], h4, dis2, jnp.pad(b4, (0, 8)))
    return out[:N]


# speedup vs baseline: 1.0607x; 1.0001x over previous
import functools

import jax
import jax.numpy as jnp
from jax import lax
from jax.experimental import pallas as pl
from jax.experimental.pallas import tpu as pltpu
from jax.experimental.pallas import tpu_sc as plsc

N = 10000
E = 160000
N_PAD = 10240
NC, NS, LANES = 2, 16, 16
NW = NC * NS
C = 128
E_PAD = 163840
CH = E_PAD // (NW * C)
CHF = E_PAD // (NS * C)
EPT = E_PAD // NW
RPT = N_PAD // NS

_MESH = plsc.VectorSubcoreMesh(core_axis_name="c", subcore_axis_name="s")
_SC_PARAMS = pltpu.CompilerParams(needs_layout_passes=False,
                                  use_tc_tiling_on_sc=False)


def _wid():
    return lax.axis_index("s") * NC + lax.axis_index("c")


@functools.partial(
    pl.kernel,
    out_type=jax.ShapeDtypeStruct((NC, N_PAD), jnp.float32),
    mesh=_MESH,
    compiler_params=_SC_PARAMS,
    scratch_types=[
        pltpu.VMEM((CH, C), jnp.float32),
        pltpu.VMEM((CH, C), jnp.int32),
        pltpu.VMEM_SHARED((N_PAD,), jnp.float32),
    ],
)
def _deg_kernel(ew3, col3, z1, out, ew_v, col_v, acc_sh):
    cid = lax.axis_index("c")
    sid = lax.axis_index("s")
    w = _wid()
    pltpu.sync_copy(ew3.at[w], ew_v)
    pltpu.sync_copy(col3.at[w], col_v)
    pltpu.sync_copy(z1.at[pl.ds(0, RPT)], acc_sh.at[pl.ds(sid * RPT, RPT)])
    plsc.subcore_barrier()

    def body(j, _):
        pltpu.sync_copy(ew_v.at[j], acc_sh.at[col_v.at[j]], add=True)
        return 0

    lax.fori_loop(0, CH, body, 0)
    plsc.subcore_barrier()
    pltpu.sync_copy(acc_sh.at[pl.ds(sid * RPT, RPT)],
                    out.at[cid, pl.ds(sid * RPT, RPT)])


@functools.partial(
    pl.kernel,
    out_type=jax.ShapeDtypeStruct((E_PAD,), jnp.float32),
    mesh=_MESH,
    compiler_params=_SC_PARAMS,
    scratch_types=[
        pltpu.VMEM((N_PAD,), jnp.float32),
        pltpu.VMEM((EPT,), jnp.int32),
        pltpu.VMEM((EPT,), jnp.int32),
        pltpu.VMEM((EPT,), jnp.float32),
        pltpu.VMEM((EPT,), jnp.float32),
    ],
)
def _norm_kernel(dis, row_f, col_f, ew_f, out, dis_v, row_v, col_v, ew_v, nrm_v):
    w = _wid()
    pltpu.sync_copy(dis, dis_v)
    pltpu.sync_copy(row_f.at[pl.ds(w * EPT, EPT)], row_v)
    pltpu.sync_copy(col_f.at[pl.ds(w * EPT, EPT)], col_v)
    pltpu.sync_copy(ew_f.at[pl.ds(w * EPT, EPT)], ew_v)

    def body(i, _):
        s = pl.ds(i * LANES, LANES)
        vr = plsc.load_gather(dis_v, [row_v[s]])
        vc = plsc.load_gather(dis_v, [col_v[s]])
        nrm_v[s] = vr * vc * ew_v[s]
        return 0

    lax.fori_loop(0, EPT // LANES, body, 0)
    pltpu.sync_copy(nrm_v, out.at[pl.ds(w * EPT, EPT)])


def _scale_chunk(buf, nrm, j, nblk):
    def body(i, _):
        nv = nrm[j, pl.ds(i * LANES, LANES)]
        for k in range(LANES):
            e = i * LANES + k
            s = nv[k]
            for db in range(nblk):
                sl = pl.ds(db * LANES, LANES)
                buf[e, sl] = buf[e, sl] * s
        return 0

    lax.fori_loop(0, C // LANES, body, 0)


def _agg_pipe(h_sh, acc_sh, ridx, cidx, nrm, g, s, t, nblk):
    n = CH

    def wait_g(b, j):
        pltpu.make_async_copy(h_sh.at[ridx.at[j]], g[b], s[b]).wait()

    def issue_g(b, j):
        pltpu.async_copy(h_sh.at[ridx.at[j]], g[b], s[b])

    def issue_s(b, j):
        pltpu.async_copy(g[b], acc_sh.at[cidx.at[j]], t[b], add=True)

    def wait_s(b, j):
        pltpu.make_async_copy(g[b], acc_sh.at[cidx.at[j]], t[b]).wait()

    def chunk(j, b, prefetch, wait_prev):
        wait_g(b, j)
        _scale_chunk(g[b], nrm, j, nblk)
        issue_s(b, j)
        if prefetch:
            bp = (b + 2) % 4
            if wait_prev:
                wait_s(bp, j - 2)
            issue_g(bp, j + 2)

    issue_g(0, 0)
    issue_g(1, 1)
    chunk(0, 0, True, False)
    chunk(1, 1, True, False)
    chunk(2, 2, True, True)
    chunk(3, 3, True, True)

    def group(ii, _):
        j0 = 4 * ii
        chunk(j0, 0, True, True)
        chunk(j0 + 1, 1, True, True)
        chunk(j0 + 2, 2, True, True)
        chunk(j0 + 3, 3, True, True)
        return 0

    g_end = (n - 6) // 4 + 1
    lax.fori_loop(1, g_end, group, 0)
    for j in range(4 * g_end, n):
        chunk(j, j % 4, j + 2 < n, True)
    for j in range(n - 4, n):
        wait_s(j % 4, j)


def _make_agg_fs(D, DG):
    D2 = D // 2

    @functools.partial(
        pl.kernel,
        out_type=jax.ShapeDtypeStruct((N_PAD, D), jnp.float32),
        mesh=_MESH,
        compiler_params=_SC_PARAMS,
        scratch_types=[
            pltpu.VMEM((CH, C), jnp.int32),
            pltpu.VMEM((CH, C), jnp.int32),
            pltpu.VMEM((CH, C), jnp.float32),
            pltpu.VMEM((C, DG), jnp.float32),
            pltpu.VMEM((C, DG), jnp.float32),
            pltpu.VMEM((C, DG), jnp.float32),
            pltpu.VMEM((C, DG), jnp.float32),
            pltpu.VMEM_SHARED((N_PAD, DG), jnp.float32),
            pltpu.VMEM_SHARED((N_PAD, D2), jnp.float32),
            pltpu.SemaphoreType.DMA,
            pltpu.SemaphoreType.DMA,
            pltpu.SemaphoreType.DMA,
            pltpu.SemaphoreType.DMA,
            pltpu.SemaphoreType.DMA,
            pltpu.SemaphoreType.DMA,
            pltpu.SemaphoreType.DMA,
            pltpu.SemaphoreType.DMA,
        ],
    )
    def agg(h, rowt, colt, nrmt, z2, out, ridx, cidx, nrm, g0, g1, g2, g3,
            h_sh, acc_sh, s0, s1, s2, s3, t0, t1, t2, t3):
        cid = lax.axis_index("c")
        sid = lax.axis_index("s")
        rows = pl.ds(sid * RPT, RPT)
        gbufs = [g0, g1, g2, g3]
        gsems = [s0, s1, s2, s3]
        tsems = [t0, t1, t2, t3]
        hsrc = h.at[rows, pl.ds(cid * DG, DG)]
        hs0 = pl.ds(0, CH)
        pltpu.async_copy(hsrc, h_sh.at[rows], s0)
        pltpu.async_copy(rowt.at[sid, hs0], ridx, s1)
        pltpu.async_copy(colt.at[sid, hs0], cidx, s2)
        pltpu.async_copy(nrmt.at[sid, hs0], nrm, s3)
        pltpu.async_copy(z2, acc_sh.at[rows], t0)
        pltpu.make_async_copy(hsrc, h_sh.at[rows], s0).wait()
        pltpu.make_async_copy(rowt.at[sid, hs0], ridx, s1).wait()
        pltpu.make_async_copy(colt.at[sid, hs0], cidx, s2).wait()
        pltpu.make_async_copy(nrmt.at[sid, hs0], nrm, s3).wait()
        pltpu.make_async_copy(z2, acc_sh.at[rows], t0).wait()
        plsc.subcore_barrier()
        _agg_pipe(h_sh, acc_sh, ridx, cidx, nrm, gbufs, gsems, tsems,
                  D2 // LANES)
        hs1 = pl.ds(CH, CH)
        pltpu.sync_copy(rowt.at[sid, hs1], ridx)
        pltpu.sync_copy(colt.at[sid, hs1], cidx)
        pltpu.sync_copy(nrmt.at[sid, hs1], nrm)
        _agg_pipe(h_sh, acc_sh, ridx, cidx, nrm, gbufs, gsems, tsems,
                  D2 // LANES)
        plsc.subcore_barrier()
        pltpu.sync_copy(acc_sh.at[rows],
                        out.at[rows, pl.ds(cid * D2, D2)])

    return agg


def _make_agg_es(D, DG):
    @functools.partial(
        pl.kernel,
        out_type=jax.ShapeDtypeStruct((NC, N_PAD, D), jnp.float32),
        mesh=_MESH,
        compiler_params=_SC_PARAMS,
        scratch_types=[
            pltpu.VMEM((CH, C), jnp.int32),
            pltpu.VMEM((CH, C), jnp.int32),
            pltpu.VMEM((CH, C), jnp.float32),
            pltpu.VMEM((C, DG), jnp.float32),
            pltpu.VMEM((C, DG), jnp.float32),
            pltpu.VMEM((C, DG), jnp.float32),
            pltpu.VMEM((C, DG), jnp.float32),
            pltpu.VMEM_SHARED((N_PAD, DG), jnp.float32),
            pltpu.VMEM_SHARED((N_PAD, D), jnp.float32),
            pltpu.SemaphoreType.DMA,
            pltpu.SemaphoreType.DMA,
            pltpu.SemaphoreType.DMA,
            pltpu.SemaphoreType.DMA,
            pltpu.SemaphoreType.DMA,
            pltpu.SemaphoreType.DMA,
            pltpu.SemaphoreType.DMA,
            pltpu.SemaphoreType.DMA,
        ],
    )
    def agg(h, rowt, colt, nrmt, z2, out, ridx, cidx, nrm, g0, g1, g2, g3,
            h_sh, acc_sh, s0, s1, s2, s3, t0, t1, t2, t3):
        cid = lax.axis_index("c")
        sid = lax.axis_index("s")
        rows = pl.ds(sid * RPT, RPT)
        w = _wid()
        hsrc = h.at[rows]
        pltpu.async_copy(hsrc, h_sh.at[rows], s0)
        pltpu.async_copy(rowt.at[w], ridx, s1)
        pltpu.async_copy(colt.at[w], cidx, s2)
        pltpu.async_copy(nrmt.at[w], nrm, s3)
        pltpu.async_copy(z2, acc_sh.at[rows], t0)
        pltpu.make_async_copy(hsrc, h_sh.at[rows], s0).wait()
        pltpu.make_async_copy(rowt.at[w], ridx, s1).wait()
        pltpu.make_async_copy(colt.at[w], cidx, s2).wait()
        pltpu.make_async_copy(nrmt.at[w], nrm, s3).wait()
        pltpu.make_async_copy(z2, acc_sh.at[rows], t0).wait()
        plsc.subcore_barrier()
        _agg_pipe(h_sh, acc_sh, ridx, cidx, nrm, [g0, g1, g2, g3],
                  [s0, s1, s2, s3], [t0, t1, t2, t3], D // LANES)
        plsc.subcore_barrier()
        pltpu.sync_copy(acc_sh.at[rows], out.at[cid, rows])

    return agg


_AGG_FS = {128: _make_agg_fs(128, 64)}
_AGG_ES = {64: _make_agg_es(64, 64), 48: _make_agg_es(48, 48)}


def _dis_body(p_ref, o_ref):
    deg = p_ref[0, :] + p_ref[1, :] + 1.0
    dis = lax.rsqrt(deg)
    o_ref[0, :] = dis
    o_ref[1, :] = 1.0 / deg


def _dis_kernel(parts):
    return pl.pallas_call(
        _dis_body,
        out_shape=jax.ShapeDtypeStruct((2, N_PAD), jnp.float32),
    )(parts)


_BM = 1024


def _pad_cols(z, ow):
    k = z.shape[1]
    if ow == k:
        return z
    zpad = jnp.zeros((z.shape[0], 8), z.dtype)
    if k == 128:
        return jnp.concatenate([z[:, :64], zpad, z[:, 64:], zpad], axis=1)
    return jnp.concatenate([z, zpad], axis=1)


def _unpad_cols(h, k):
    if h.shape[1] == k:
        return h
    if k == 128:
        return jnp.concatenate([h[:, :64], h[:, 72:136]], axis=1)
    return h[:, :k]


def _mm_body(ow, x_ref, w_ref, o_ref):
    z = jnp.dot(x_ref[...], w_ref[...], preferred_element_type=jnp.float32)
    o_ref[...] = _pad_cols(z, ow)


def _matmul(x, W, ow):
    M, K = x.shape
    D = W.shape[1]
    return pl.pallas_call(
        functools.partial(_mm_body, ow),
        grid=(M // _BM,),
        in_specs=[pl.BlockSpec((_BM, K), lambda i: (i, 0)),
                  pl.BlockSpec((K, D), lambda i: (0, 0))],
        out_specs=pl.BlockSpec((_BM, ow), lambda i: (i, 0)),
        out_shape=jax.ShapeDtypeStruct((M, ow), jnp.float32),
    )(x, W)


def _comb1_mm_body(ow, p_ref, h_ref, d2_ref, b_ref, w_ref, o_ref):
    k = p_ref.shape[1]
    a = p_ref[...] + d2_ref[...] * _unpad_cols(h_ref[...], k) + b_ref[...]
    a = jnp.maximum(a, 0.0)
    z = jnp.dot(a, w_ref[...], preferred_element_type=jnp.float32)
    o_ref[...] = _pad_cols(z, ow)


def _combine1_matmul(p, h, dis2, b, W, ow):
    M, K = p.shape
    KH = h.shape[1]
    D = W.shape[1]
    return pl.pallas_call(
        functools.partial(_comb1_mm_body, ow),
        grid=(M // _BM,),
        in_specs=[pl.BlockSpec((_BM, K), lambda i: (i, 0)),
                  pl.BlockSpec((_BM, KH), lambda i: (i, 0)),
                  pl.BlockSpec((_BM, 1), lambda i: (i, 0)),
                  pl.BlockSpec((1, K), lambda i: (0, 0)),
                  pl.BlockSpec((K, D), lambda i: (0, 0))],
        out_specs=pl.BlockSpec((_BM, ow), lambda i: (i, 0)),
        out_shape=jax.ShapeDtypeStruct((M, ow), jnp.float32),
    )(p, h, dis2, b.reshape(1, K), W)


def _comb2_mm_body(ow, p0_ref, p1_ref, h_ref, d2_ref, b_ref, w_ref, o_ref):
    k = p0_ref.shape[1]
    a = (p0_ref[...] + p1_ref[...]
         + d2_ref[...] * _unpad_cols(h_ref[...], k) + b_ref[...])
    a = jnp.maximum(a, 0.0)
    z = jnp.dot(a, w_ref[...], preferred_element_type=jnp.float32)
    o_ref[...] = _pad_cols(z, ow)


def _combine2_matmul(p0, p1, h, dis2, b, W, ow):
    M, K = p0.shape
    KH = h.shape[1]
    D = W.shape[1]
    blk = pl.BlockSpec((_BM, K), lambda i: (i, 0))
    return pl.pallas_call(
        functools.partial(_comb2_mm_body, ow),
        grid=(M // _BM,),
        in_specs=[blk, blk,
                  pl.BlockSpec((_BM, KH), lambda i: (i, 0)),
                  pl.BlockSpec((_BM, 1), lambda i: (i, 0)),
                  pl.BlockSpec((1, K), lambda i: (0, 0)),
                  pl.BlockSpec((K, D), lambda i: (0, 0))],
        out_specs=pl.BlockSpec((_BM, ow), lambda i: (i, 0)),
        out_shape=jax.ShapeDtypeStruct((M, ow), jnp.float32),
    )(p0, p1, h, dis2, b.reshape(1, K), W)


def _final_body(p0_ref, p1_ref, h_ref, d2_ref, b_ref, o_ref):
    z = p0_ref[...] + p1_ref[...] + d2_ref[...] * h_ref[...] + b_ref[...]
    mask = lax.broadcasted_iota(jnp.int32, z.shape, 1) < 40
    zm = jnp.where(mask, z, -jnp.inf)
    mx = jnp.max(zm, axis=1, keepdims=True)
    ex = jnp.where(mask, jnp.exp(z - mx), 0.0)
    lse = jnp.log(jnp.sum(ex, axis=1, keepdims=True))
    o_ref[...] = (z - mx - lse)[:, :40]


def _final(p0, p1, h, dis2, b):
    M, K = h.shape
    blk = pl.BlockSpec((_BM, K), lambda i: (i, 0))
    return pl.pallas_call(
        _final_body,
        grid=(M // _BM,),
        in_specs=[blk, blk, blk,
                  pl.BlockSpec((_BM, 1), lambda i: (i, 0)),
                  pl.BlockSpec((1, K), lambda i: (0, 0))],
        out_specs=pl.BlockSpec((_BM, 40), lambda i: (i, 0)),
        out_shape=jax.ShapeDtypeStruct((M, 40), jnp.float32),
    )(p0, p1, h, dis2, b.reshape(1, K))


def kernel(x, edge_index, edge_attr, W1, b1, W2, b2, W3, b3, W4, b4):
    pad = E_PAD - E
    row = jnp.concatenate([edge_index[0].astype(jnp.int32),
                           jnp.zeros((pad,), jnp.int32)])
    col = jnp.concatenate([edge_index[1].astype(jnp.int32),
                           jnp.zeros((pad,), jnp.int32)])
    ew = jnp.concatenate([edge_attr.astype(jnp.float32),
                          jnp.zeros((pad,), jnp.float32)])
    row3 = row.reshape(NW, CH, C)
    col3 = col.reshape(NW, CH, C)
    ew3 = ew.reshape(NW, CH, C)
    rowf = row.reshape(NS, CHF, C)
    colf = col.reshape(NS, CHF, C)
    z1 = jnp.zeros((N_PAD,), jnp.float32)
    xp = jnp.pad(x, ((0, N_PAD - N), (0, 0)))

    parts = _deg_kernel(ew3, col3, z1)
    dd = _dis_kernel(parts)
    dis2 = dd[1].reshape(N_PAD, 1)

    nrm = _norm_kernel(dd[0], row, col, ew)
    nrm3 = nrm.reshape(NW, CH, C)
    nrmf = nrm.reshape(NS, CHF, C)

    z64 = jnp.zeros((RPT, 64), jnp.float32)
    z48 = jnp.zeros((RPT, 48), jnp.float32)

    h1 = _matmul(xp, W1, 128)
    p = _AGG_FS[128](h1, rowf, colf, nrmf, z64)
    h2 = _combine1_matmul(p, h1, dis2, b1, W2, 128)
    p = _AGG_FS[128](h2, rowf, colf, nrmf, z64)
    h3 = _combine1_matmul(p, h2, dis2, b2, W3, 64)
    p = _AGG_ES[64](h3, row3, col3, nrm3, z64)
    W4p = jnp.pad(W4, ((0, 0), (0, 8)))
    h4 = _combine2_matmul(p[0], p[1], h3, dis2, b3, W4p, 48)
    p = _AGG_ES[48](h4, row3, col3, nrm3, z48)
    out = _final(p[0], p[1], h4, dis2, jnp.pad(b4, (0, 8)))
    return out[:N]
